# baseline (speedup 1.0000x reference)
import functools

import jax
import jax.numpy as jnp
from jax import lax
from jax.experimental import pallas as pl
from jax.experimental.pallas import tpu as pltpu
from jax.experimental.pallas import tpu_sc as plsc

_N = 10000
_E = 320000
_G = 512
_D = 128

_NC, _NS = 2, 16
_NW = _NC * _NS

_mesh = plsc.VectorSubcoreMesh(core_axis_name="c", subcore_axis_name="s")

_CA = 400
_EPW = _E // _NW
_NCHA = _EPW // _CA


@functools.partial(
    pl.kernel,
    mesh=_mesh,
    out_type=jax.ShapeDtypeStruct((_N, _D), jnp.float32),
    scratch_types=[
        pltpu.VMEM((_CA,), jnp.int32),
        pltpu.VMEM((_CA, _D), jnp.float32),
        pltpu.SemaphoreType.DMA,
    ],
)
def _sc_gather_ub(u_hbm, batch_hbm, ub_out, bidx, buf, sem):
    wid = lax.axis_index("c") * _NS + lax.axis_index("s")

    @pl.when(wid < _N // _CA)
    def _():
        nb = pl.multiple_of(wid * _CA, 8)
        pltpu.sync_copy(batch_hbm.at[pl.ds(nb, _CA)], bidx)
        pltpu.async_copy(u_hbm.at[bidx], buf, sem).wait()
        pltpu.sync_copy(buf, ub_out.at[pl.ds(nb, _CA)])


@functools.partial(
    pl.kernel,
    mesh=_mesh,
    out_type=[
        jax.ShapeDtypeStruct((_E, _D), jnp.float32),
        jax.ShapeDtypeStruct((_E, _D), jnp.float32),
        jax.ShapeDtypeStruct((_E, _D), jnp.float32),
    ],
    scratch_types=[
        pltpu.VMEM((_CA,), jnp.int32),
        pltpu.VMEM((_CA,), jnp.int32),
        pltpu.VMEM((_CA, _D), jnp.float32),
        pltpu.VMEM((_CA, _D), jnp.float32),
        pltpu.SemaphoreType.DMA,
        pltpu.SemaphoreType.DMA,
    ],
)
def _sc_gather_edges(x_hbm, ub_hbm, row_hbm, col_hbm,
                     xr_out, xc_out, ue_out,
                     ridx, cidx, buf_a, buf_b, sem1, sem2):
    wid = lax.axis_index("c") * _NS + lax.axis_index("s")
    ebase = wid * _EPW

    def chunk(i, carry):
        b = pl.multiple_of(ebase + i * _CA, 8)
        pltpu.sync_copy(row_hbm.at[pl.ds(b, _CA)], ridx)
        pltpu.sync_copy(col_hbm.at[pl.ds(b, _CA)], cidx)
        cp_a = pltpu.async_copy(x_hbm.at[ridx], buf_a, sem1)
        cp_b = pltpu.async_copy(x_hbm.at[cidx], buf_b, sem2)
        cp_a.wait()
        cp_b.wait()
        pltpu.sync_copy(buf_a, xr_out.at[pl.ds(b, _CA)])
        pltpu.sync_copy(buf_b, xc_out.at[pl.ds(b, _CA)])
        pltpu.async_copy(ub_hbm.at[ridx], buf_a, sem1).wait()
        pltpu.sync_copy(buf_a, ue_out.at[pl.ds(b, _CA)])
        return carry

    lax.fori_loop(0, _NCHA, chunk, 0)


_TB = 3200


def _leaky(h):
    return jnp.where(h >= 0, h, 0.01 * h)


def _edge_body(xr, xc, ea, ue, w1a, w1b, w1c, b1, w2, b2,
               we1, be1, we2, be2, en_out, eo_out):
    bf = jnp.bfloat16
    s = (xr[...] + xc[...]).astype(bf)
    h = jnp.dot(s, w1a[...], preferred_element_type=jnp.float32)
    h = h + jnp.dot(ea[...].astype(bf), w1b[...], preferred_element_type=jnp.float32)
    h = h + jnp.dot(ue[...].astype(bf), w1c[...], preferred_element_type=jnp.float32)
    h = _leaky(h + b1[...]).astype(bf)
    en = _leaky(jnp.dot(h, w2[...], preferred_element_type=jnp.float32) + b2[...])
    en_out[...] = en
    h2 = _leaky(jnp.dot(en.astype(bf), we1[...], preferred_element_type=jnp.float32) + be1[...]).astype(bf)
    eo_out[...] = ea[...] + jnp.dot(h2, we2[...], preferred_element_type=jnp.float32) + be2[...]


def _tc_edge(xr, xc, ea, ue, wp, ep):
    w1, b1, w2, b2 = wp
    we1, be1, we2, be2 = ep
    full = lambda shape: pl.BlockSpec(shape, lambda i: (0, 0))
    return pl.pallas_call(
        _edge_body,
        grid=(_E // _TB,),
        in_specs=[
            pl.BlockSpec((_TB, _D), lambda i: (i, 0)),
            pl.BlockSpec((_TB, _D), lambda i: (i, 0)),
            pl.BlockSpec((_TB, _D), lambda i: (i, 0)),
            pl.BlockSpec((_TB, _D), lambda i: (i, 0)),
            full((_D, 768)), full((_D, 768)), full((_D, 768)), full((1, 768)),
            full((768, _D)), full((1, _D)),
            full((_D, 256)), full((1, 256)), full((256, _D)), full((1, _D)),
        ],
        out_specs=[
            pl.BlockSpec((_TB, _D), lambda i: (i, 0)),
            pl.BlockSpec((_TB, _D), lambda i: (i, 0)),
        ],
        out_shape=[
            jax.ShapeDtypeStruct((_E, _D), jnp.float32),
            jax.ShapeDtypeStruct((_E, _D), jnp.float32),
        ],
        compiler_params=pltpu.CompilerParams(
            dimension_semantics=("arbitrary",)),
    )(xr, xc, ea, ue,
      w1[0:_D].astype(jnp.bfloat16), w1[_D:2 * _D].astype(jnp.bfloat16),
      w1[2 * _D:3 * _D].astype(jnp.bfloat16), b1.reshape(1, 768),
      w2.astype(jnp.bfloat16), b2.reshape(1, _D),
      we1.astype(jnp.bfloat16), be1.reshape(1, 256),
      we2.astype(jnp.bfloat16), be2.reshape(1, _D))


_CC = 128
_NR = _N // _NC
_SHN = 5120
_ZR = _SHN // _NS
_FP = 40
_NP = _ZR // _FP
_NCHC = _E // _CC
_ITC = 157


@functools.partial(
    pl.kernel,
    mesh=_mesh,
    out_type=[
        jax.ShapeDtypeStruct((_N, _D), jnp.float32),
        jax.ShapeDtypeStruct((_N, _D), jnp.float32),
        jax.ShapeDtypeStruct((_N, _D), jnp.float32),
    ],
    scratch_types=[
        pltpu.VMEM((_CC,), jnp.int32),
        pltpu.VMEM((_CC,), jnp.int32),
        pltpu.VMEM((_CC, _D), jnp.float32),
        pltpu.VMEM((_CC, _D), jnp.float32),
        pltpu.VMEM((_FP, _D), jnp.float32),
        pltpu.VMEM((_FP, _D), jnp.float32),
        pltpu.VMEM_SHARED((_SHN, _D), jnp.float32),
        pltpu.VMEM_SHARED((_SHN, _D), jnp.float32),
    ],
)
def _sc_scatter_e(enew_hbm, row_hbm, zrow_hbm, ones_hbm,
                  agg_out, nsum_out, ncnt_out,
                  ridx, lidx, vals, ones_v, fbuf, cbuf, ssum, scnt):
    c = lax.axis_index("c")
    t = lax.axis_index("s")
    nbase = c * _NR
    sb = pl.multiple_of(t * _ZR, 8)
    pltpu.sync_copy(zrow_hbm, ssum.at[pl.ds(sb, _ZR)])
    pltpu.sync_copy(zrow_hbm, scnt.at[pl.ds(sb, _ZR)])
    pltpu.sync_copy(ones_hbm, ones_v)
    plsc.subcore_barrier()

    def chunk(i, carry):
        cid = t + _NS * i

        @pl.when(cid < _NCHC)
        def _():
            b = pl.multiple_of(cid * _CC, 8)
            pltpu.sync_copy(row_hbm.at[pl.ds(b, _CC)], ridx)

            def ixl(j, c2):
                v = ridx[pl.ds(j * 16, 16)] - nbase
                ok = (v >= 0) & (v < _NR)
                lidx[pl.ds(j * 16, 16)] = jnp.where(ok, v, _NR)
                return c2
            lax.fori_loop(0, _CC // 16, ixl, 0)
            pltpu.sync_copy(enew_hbm.at[pl.ds(b, _CC)], vals)
            pltpu.sync_copy(vals, ssum.at[lidx], add=True)
            pltpu.sync_copy(ones_v, scnt.at[lidx], add=True)
        return carry

    lax.fori_loop(0, _ITC, chunk, 0)
    plsc.subcore_barrier()

    def piece(p, carry):
        @pl.when((t < _NS - 1) | (p < (_NR - (_NS - 1) * _ZR) // _FP))
        def _():
            lb = pl.multiple_of(sb + p * _FP, 8)
            gb = pl.multiple_of(nbase + sb + p * _FP, 8)
            pltpu.sync_copy(ssum.at[pl.ds(lb, _FP)], fbuf)
            pltpu.sync_copy(scnt.at[pl.ds(lb, _FP)], cbuf)
            pltpu.sync_copy(fbuf, nsum_out.at[pl.ds(gb, _FP)])
            pltpu.sync_copy(cbuf, ncnt_out.at[pl.ds(gb, _FP)])

            def divloop(r, c2):
                inv = 1.0 / jnp.maximum(cbuf[r, pl.ds(0, 16)], 1.0)

                def dj(j, c3):
                    fbuf[r, pl.ds(j * 16, 16)] = fbuf[r, pl.ds(j * 16, 16)] * inv
                    return c3
                lax.fori_loop(0, _D // 16, dj, 0)
                return c2
            lax.fori_loop(0, _FP, divloop, 0)
            pltpu.sync_copy(fbuf, agg_out.at[pl.ds(gb, _FP)])
        return carry
    lax.fori_loop(0, _NP, piece, 0)


_NB = 400


def _node_body(x, agg, ub, w1a, w1b, w1c, b1, w2, b2,
               wv1, bv1, wv2, bv2, xn_out, xo_out):
    h = jnp.dot(x[...], w1a[...], preferred_element_type=jnp.float32)
    h = h + jnp.dot(agg[...], w1b[...], preferred_element_type=jnp.float32)
    h = h + jnp.dot(ub[...], w1c[...], preferred_element_type=jnp.float32)
    h = _leaky(h + b1[...])
    xn = _leaky(jnp.dot(h, w2[...], preferred_element_type=jnp.float32) + b2[...])
    xn_out[...] = xn
    h2 = _leaky(jnp.dot(xn, wv1[...], preferred_element_type=jnp.float32) + bv1[...])
    xo_out[...] = x[...] + jnp.dot(h2, wv2[...], preferred_element_type=jnp.float32) + bv2[...]


def _tc_node(x, agg, ub, wp, vp):
    w1, b1, w2, b2 = wp
    wv1, bv1, wv2, bv2 = vp
    full = lambda shape: pl.BlockSpec(shape, lambda i: (0, 0))
    return pl.pallas_call(
        _node_body,
        grid=(_N // _NB,),
        in_specs=[
            pl.BlockSpec((_NB, _D), lambda i: (i, 0)),
            pl.BlockSpec((_NB, _D), lambda i: (i, 0)),
            pl.BlockSpec((_NB, _D), lambda i: (i, 0)),
            full((_D, 768)), full((_D, 768)), full((_D, 768)), full((1, 768)),
            full((768, _D)), full((1, _D)),
            full((_D, 256)), full((1, 256)), full((256, _D)), full((1, _D)),
        ],
        out_specs=[
            pl.BlockSpec((_NB, _D), lambda i: (i, 0)),
            pl.BlockSpec((_NB, _D), lambda i: (i, 0)),
        ],
        out_shape=[
            jax.ShapeDtypeStruct((_N, _D), jnp.float32),
            jax.ShapeDtypeStruct((_N, _D), jnp.float32),
        ],
        compiler_params=pltpu.CompilerParams(
            dimension_semantics=("arbitrary",)),
    )(x, agg, ub,
      w1[0:_D], w1[_D:2 * _D], w1[2 * _D:3 * _D], b1.reshape(1, 768),
      w2, b2.reshape(1, _D),
      wv1, bv1.reshape(1, 256), wv2, bv2.reshape(1, _D))


_CE = 80
_GR = _G // _NC
_SHG = 512
_NCHE = _N // _CE
_ITE = 8
_GZ = _SHG // _NS


@functools.partial(
    pl.kernel,
    mesh=_mesh,
    out_type=[
        jax.ShapeDtypeStruct((_G, _D), jnp.float32),
        jax.ShapeDtypeStruct((_G, _D), jnp.float32),
    ],
    scratch_types=[
        pltpu.VMEM((_CE,), jnp.int32),
        pltpu.VMEM((_CE,), jnp.int32),
        pltpu.VMEM((_CE, _D), jnp.float32),
        pltpu.VMEM((_CE, _D), jnp.float32),
        pltpu.VMEM((_CE, _D), jnp.float32),
        pltpu.VMEM((_CE, _D), jnp.float32),
        pltpu.VMEM((_GZ, _D), jnp.float32),
        pltpu.VMEM((_GZ, _D), jnp.float32),
        pltpu.VMEM_SHARED((_SHG, _D), jnp.float32),
        pltpu.VMEM_SHARED((_SHG, _D), jnp.float32),
        pltpu.VMEM_SHARED((_SHG, _D), jnp.float32),
        pltpu.VMEM_SHARED((_SHG, _D), jnp.float32),
    ],
)
def _sc_scatter_g(xnew_hbm, nsum_hbm, ncnt_hbm, batch_hbm,
                  zrow_hbm, ones_hbm,
                  gx_out, ge_out,
                  bidx, lidx, xv, sv, cv, ones_v, gbuf, cbuf,
                  sgx, sge, sgxc, sgec):
    c = lax.axis_index("c")
    t = lax.axis_index("s")
    gbase = c * _GR
    zb = pl.multiple_of(t * _GZ, 8)
    pltpu.sync_copy(zrow_hbm, sgx.at[pl.ds(zb, _GZ)])
    pltpu.sync_copy(zrow_hbm, sge.at[pl.ds(zb, _GZ)])
    pltpu.sync_copy(zrow_hbm, sgxc.at[pl.ds(zb, _GZ)])
    pltpu.sync_copy(zrow_hbm, sgec.at[pl.ds(zb, _GZ)])
    pltpu.sync_copy(ones_hbm, ones_v)
    plsc.subcore_barrier()

    def chunk(i, carry):
        cid = t + _NS * i

        @pl.when(cid < _NCHE)
        def _():
            b = pl.multiple_of(cid * _CE, 8)
            pltpu.sync_copy(batch_hbm.at[pl.ds(b, _CE)], bidx)

            def ixl(j, c2):
                v = bidx[pl.ds(j * 16, 16)] - gbase
                ok = (v >= 0) & (v < _GR)
                lidx[pl.ds(j * 16, 16)] = jnp.where(ok, v, _GR)
                return c2
            lax.fori_loop(0, _CE // 16, ixl, 0)
            pltpu.sync_copy(xnew_hbm.at[pl.ds(b, _CE)], xv)
            pltpu.sync_copy(nsum_hbm.at[pl.ds(b, _CE)], sv)
            pltpu.sync_copy(ncnt_hbm.at[pl.ds(b, _CE)], cv)
            pltpu.sync_copy(xv, sgx.at[lidx], add=True)
            pltpu.sync_copy(sv, sge.at[lidx], add=True)
            pltpu.sync_copy(ones_v, sgxc.at[lidx], add=True)
            pltpu.sync_copy(cv, sgec.at[lidx], add=True)
        return carry

    lax.fori_loop(0, _ITE, chunk, 0)
    plsc.subcore_barrier()

    @pl.when(t < _GR // _GZ)
    def _():
        gb = pl.multiple_of(gbase + t * _GZ, 8)

        def final(src, cnts, out):
            pltpu.sync_copy(src.at[pl.ds(zb, _GZ)], gbuf)
            pltpu.sync_copy(cnts.at[pl.ds(zb, _GZ)], cbuf)

            def divloop(r, carry):
                inv = 1.0 / jnp.maximum(cbuf[r, pl.ds(0, 16)], 1.0)

                def dj(j, c2):
                    gbuf[r, pl.ds(j * 16, 16)] = gbuf[r, pl.ds(j * 16, 16)] * inv
                    return c2
                lax.fori_loop(0, _D // 16, dj, 0)
                return carry
            lax.fori_loop(0, _GZ, divloop, 0)
            pltpu.sync_copy(gbuf, out.at[pl.ds(gb, _GZ)])

        final(sgx, sgxc, gx_out)
        final(sge, sgec, ge_out)


def _glob_body(u, gx, ge, w1a, w1b, w1c, b1, w2, b2,
               wu1, bu1, wu2, bu2, uo_out):
    h = jnp.dot(u[...], w1a[...], preferred_element_type=jnp.float32)
    h = h + jnp.dot(gx[...], w1b[...], preferred_element_type=jnp.float32)
    h = h + jnp.dot(ge[...], w1c[...], preferred_element_type=jnp.float32)
    h = _leaky(h + b1[...])
    un = _leaky(jnp.dot(h, w2[...], preferred_element_type=jnp.float32) + b2[...])
    h2 = _leaky(jnp.dot(un, wu1[...], preferred_element_type=jnp.float32) + bu1[...])
    uo_out[...] = u[...] + jnp.dot(h2, wu2[...], preferred_element_type=jnp.float32) + bu2[...]


def _tc_glob(u, gx, ge, wp, up):
    w1, b1, w2, b2 = wp
    wu1, bu1, wu2, bu2 = up
    return pl.pallas_call(
        _glob_body,
        out_shape=jax.ShapeDtypeStruct((_G, _D), jnp.float32),
    )(u, gx, ge,
      w1[0:_D], w1[_D:2 * _D], w1[2 * _D:3 * _D], b1.reshape(1, 768),
      w2, b2.reshape(1, _D),
      wu1, bu1.reshape(1, 256), wu2, bu2.reshape(1, _D))


def kernel(x, edge_index, edge_attr, u, batch, params):
    row = edge_index[0].astype(jnp.int32)
    col = edge_index[1].astype(jnp.int32)
    batch32 = batch.astype(jnp.int32)

    ub = _sc_gather_ub(u, batch32)
    xr, xc, ue = _sc_gather_edges(x, ub, row, col)
    e_new, e_out = _tc_edge(xr, xc, edge_attr, ue, params['edge'], params['e'])

    zrow = jnp.zeros((_ZR, _D), jnp.float32)
    ones_c = jnp.ones((_CC, _D), jnp.float32)
    agg, nsum, ncnt = _sc_scatter_e(e_new, row, zrow, ones_c)

    x_new, x_out = _tc_node(x, agg, ub, params['node'], params['v'])

    zrow_g = jnp.zeros((_GZ, _D), jnp.float32)
    ones_e = jnp.ones((_CE, _D), jnp.float32)
    gx, ge = _sc_scatter_g(x_new, nsum, ncnt, batch32, zrow_g, ones_e)

    u_out = _tc_glob(u, gx, ge, params['glob'], params['u'])
    return (x_out, e_out, u_out)


# speedup vs baseline: 1.0142x; 1.0142x over previous
import functools

import jax
import jax.numpy as jnp
from jax import lax
from jax.experimental import pallas as pl
from jax.experimental.pallas import tpu as pltpu
from jax.experimental.pallas import tpu_sc as plsc

_N = 10000
_E = 320000
_G = 512
_D = 128

_NC, _NS = 2, 16
_NW = _NC * _NS

_mesh = plsc.VectorSubcoreMesh(core_axis_name="c", subcore_axis_name="s")

_CA = 400
_E2 = _E // 2
_CAH = 200
_EPWH = _E2 // _NW
_NCHAH = _EPWH // _CAH


@functools.partial(
    pl.kernel,
    mesh=_mesh,
    out_type=jax.ShapeDtypeStruct((_N, _D), jnp.float32),
    scratch_types=[
        pltpu.VMEM((_CA,), jnp.int32),
        pltpu.VMEM((_CA, _D), jnp.float32),
        pltpu.SemaphoreType.DMA,
    ],
)
def _sc_gather_ub(u_hbm, batch_hbm, ub_out, bidx, buf, sem):
    wid = lax.axis_index("c") * _NS + lax.axis_index("s")

    @pl.when(wid < _N // _CA)
    def _():
        nb = pl.multiple_of(wid * _CA, 8)
        pltpu.sync_copy(batch_hbm.at[pl.ds(nb, _CA)], bidx)
        pltpu.async_copy(u_hbm.at[bidx], buf, sem).wait()
        pltpu.sync_copy(buf, ub_out.at[pl.ds(nb, _CA)])


@functools.partial(
    pl.kernel,
    mesh=_mesh,
    out_type=[
        jax.ShapeDtypeStruct((_E2, _D), jnp.float32),
        jax.ShapeDtypeStruct((_E2, _D), jnp.float32),
        jax.ShapeDtypeStruct((_E2, _D), jnp.float32),
    ],
    scratch_types=[
        pltpu.VMEM((_CAH,), jnp.int32),
        pltpu.VMEM((_CAH,), jnp.int32),
        pltpu.VMEM((_CAH, _D), jnp.float32),
        pltpu.VMEM((_CAH, _D), jnp.float32),
        pltpu.SemaphoreType.DMA,
        pltpu.SemaphoreType.DMA,
    ],
)
def _sc_gather_edges(x_hbm, ub_hbm, row_hbm, col_hbm,
                     xr_out, xc_out, ue_out,
                     ridx, cidx, buf_a, buf_b, sem1, sem2):
    wid = lax.axis_index("c") * _NS + lax.axis_index("s")
    ebase = wid * _EPWH

    def chunk(i, carry):
        b = pl.multiple_of(ebase + i * _CAH, 8)
        pltpu.sync_copy(row_hbm.at[pl.ds(b, _CAH)], ridx)
        pltpu.sync_copy(col_hbm.at[pl.ds(b, _CAH)], cidx)
        cp_a = pltpu.async_copy(x_hbm.at[ridx], buf_a, sem1)
        cp_b = pltpu.async_copy(x_hbm.at[cidx], buf_b, sem2)
        cp_a.wait()
        cp_b.wait()
        pltpu.sync_copy(buf_a, xr_out.at[pl.ds(b, _CAH)])
        pltpu.sync_copy(buf_b, xc_out.at[pl.ds(b, _CAH)])
        pltpu.async_copy(ub_hbm.at[ridx], buf_a, sem1).wait()
        pltpu.sync_copy(buf_a, ue_out.at[pl.ds(b, _CAH)])
        return carry

    lax.fori_loop(0, _NCHAH, chunk, 0)


_TB = 3200


def _leaky(h):
    return jnp.where(h >= 0, h, 0.01 * h)


def _edge_body(xr, xc, ea, ue, w1a, w1b, w1c, b1, w2, b2,
               we1, be1, we2, be2, en_out, eo_out):
    bf = jnp.bfloat16
    s = (xr[...] + xc[...]).astype(bf)
    h = jnp.dot(s, w1a[...], preferred_element_type=jnp.float32)
    h = h + jnp.dot(ea[...].astype(bf), w1b[...], preferred_element_type=jnp.float32)
    h = h + jnp.dot(ue[...].astype(bf), w1c[...], preferred_element_type=jnp.float32)
    h = _leaky(h + b1[...]).astype(bf)
    en = _leaky(jnp.dot(h, w2[...], preferred_element_type=jnp.float32) + b2[...])
    en_out[...] = en
    h2 = _leaky(jnp.dot(en.astype(bf), we1[...], preferred_element_type=jnp.float32) + be1[...]).astype(bf)
    eo_out[...] = ea[...] + jnp.dot(h2, we2[...], preferred_element_type=jnp.float32) + be2[...]


def _tc_edge(xr, xc, ea, ue, wp, ep):
    w1, b1, w2, b2 = wp
    we1, be1, we2, be2 = ep
    full = lambda shape: pl.BlockSpec(shape, lambda i: (0, 0))
    return pl.pallas_call(
        _edge_body,
        grid=(_E2 // _TB,),
        in_specs=[
            pl.BlockSpec((_TB, _D), lambda i: (i, 0)),
            pl.BlockSpec((_TB, _D), lambda i: (i, 0)),
            pl.BlockSpec((_TB, _D), lambda i: (i, 0)),
            pl.BlockSpec((_TB, _D), lambda i: (i, 0)),
            full((_D, 768)), full((_D, 768)), full((_D, 768)), full((1, 768)),
            full((768, _D)), full((1, _D)),
            full((_D, 256)), full((1, 256)), full((256, _D)), full((1, _D)),
        ],
        out_specs=[
            pl.BlockSpec((_TB, _D), lambda i: (i, 0)),
            pl.BlockSpec((_TB, _D), lambda i: (i, 0)),
        ],
        out_shape=[
            jax.ShapeDtypeStruct((_E2, _D), jnp.float32),
            jax.ShapeDtypeStruct((_E2, _D), jnp.float32),
        ],
        compiler_params=pltpu.CompilerParams(
            dimension_semantics=("arbitrary",)),
    )(xr, xc, ea, ue,
      w1[0:_D].astype(jnp.bfloat16), w1[_D:2 * _D].astype(jnp.bfloat16),
      w1[2 * _D:3 * _D].astype(jnp.bfloat16), b1.reshape(1, 768),
      w2.astype(jnp.bfloat16), b2.reshape(1, _D),
      we1.astype(jnp.bfloat16), be1.reshape(1, 256),
      we2.astype(jnp.bfloat16), be2.reshape(1, _D))


_CC = 128
_NR = _N // _NC
_SHN = 5120
_ZR = _SHN // _NS
_FP = 40
_NP = _ZR // _FP
_NCHC = _E // _CC
_ITC = 157


@functools.partial(
    pl.kernel,
    mesh=_mesh,
    out_type=[
        jax.ShapeDtypeStruct((_N, _D), jnp.float32),
        jax.ShapeDtypeStruct((_N, _D), jnp.float32),
        jax.ShapeDtypeStruct((_N, _D), jnp.float32),
    ],
    scratch_types=[
        pltpu.VMEM((_CC,), jnp.int32),
        pltpu.VMEM((_CC,), jnp.int32),
        pltpu.VMEM((_CC, _D), jnp.float32),
        pltpu.VMEM((_CC, _D), jnp.float32),
        pltpu.VMEM((_FP, _D), jnp.float32),
        pltpu.VMEM((_FP, _D), jnp.float32),
        pltpu.VMEM_SHARED((_SHN, _D), jnp.float32),
        pltpu.VMEM_SHARED((_SHN, _D), jnp.float32),
    ],
)
def _sc_scatter_e(enew0_hbm, enew1_hbm, row_hbm, zrow_hbm, ones_hbm,
                  agg_out, nsum_out, ncnt_out,
                  ridx, lidx, vals, ones_v, fbuf, cbuf, ssum, scnt):
    c = lax.axis_index("c")
    t = lax.axis_index("s")
    nbase = c * _NR
    sb = pl.multiple_of(t * _ZR, 8)
    pltpu.sync_copy(zrow_hbm, ssum.at[pl.ds(sb, _ZR)])
    pltpu.sync_copy(zrow_hbm, scnt.at[pl.ds(sb, _ZR)])
    pltpu.sync_copy(ones_hbm, ones_v)
    plsc.subcore_barrier()

    def chunk(i, carry):
        cid = t + _NS * i

        @pl.when(cid < _NCHC)
        def _():
            b = pl.multiple_of(cid * _CC, 8)
            pltpu.sync_copy(row_hbm.at[pl.ds(b, _CC)], ridx)

            def ixl(j, c2):
                v = ridx[pl.ds(j * 16, 16)] - nbase
                ok = (v >= 0) & (v < _NR)
                lidx[pl.ds(j * 16, 16)] = jnp.where(ok, v, _NR)
                return c2
            lax.fori_loop(0, _CC // 16, ixl, 0)

            @pl.when(b < _E2)
            def _():
                pltpu.sync_copy(enew0_hbm.at[pl.ds(b, _CC)], vals)

            @pl.when(b >= _E2)
            def _():
                b2 = pl.multiple_of(b - _E2, 8)
                pltpu.sync_copy(enew1_hbm.at[pl.ds(b2, _CC)], vals)

            pltpu.sync_copy(vals, ssum.at[lidx], add=True)
            pltpu.sync_copy(ones_v, scnt.at[lidx], add=True)
        return carry

    lax.fori_loop(0, _ITC, chunk, 0)
    plsc.subcore_barrier()

    def piece(p, carry):
        @pl.when((t < _NS - 1) | (p < (_NR - (_NS - 1) * _ZR) // _FP))
        def _():
            lb = pl.multiple_of(sb + p * _FP, 8)
            gb = pl.multiple_of(nbase + sb + p * _FP, 8)
            pltpu.sync_copy(ssum.at[pl.ds(lb, _FP)], fbuf)
            pltpu.sync_copy(scnt.at[pl.ds(lb, _FP)], cbuf)
            pltpu.sync_copy(fbuf, nsum_out.at[pl.ds(gb, _FP)])
            pltpu.sync_copy(cbuf, ncnt_out.at[pl.ds(gb, _FP)])

            def divloop(r, c2):
                inv = 1.0 / jnp.maximum(cbuf[r, pl.ds(0, 16)], 1.0)

                def dj(j, c3):
                    fbuf[r, pl.ds(j * 16, 16)] = fbuf[r, pl.ds(j * 16, 16)] * inv
                    return c3
                lax.fori_loop(0, _D // 16, dj, 0)
                return c2
            lax.fori_loop(0, _FP, divloop, 0)
            pltpu.sync_copy(fbuf, agg_out.at[pl.ds(gb, _FP)])
        return carry
    lax.fori_loop(0, _NP, piece, 0)


_NB = 400


def _node_body(x, agg, ub, w1a, w1b, w1c, b1, w2, b2,
               wv1, bv1, wv2, bv2, xn_out, xo_out):
    h = jnp.dot(x[...], w1a[...], preferred_element_type=jnp.float32)
    h = h + jnp.dot(agg[...], w1b[...], preferred_element_type=jnp.float32)
    h = h + jnp.dot(ub[...], w1c[...], preferred_element_type=jnp.float32)
    h = _leaky(h + b1[...])
    xn = _leaky(jnp.dot(h, w2[...], preferred_element_type=jnp.float32) + b2[...])
    xn_out[...] = xn
    h2 = _leaky(jnp.dot(xn, wv1[...], preferred_element_type=jnp.float32) + bv1[...])
    xo_out[...] = x[...] + jnp.dot(h2, wv2[...], preferred_element_type=jnp.float32) + bv2[...]


def _tc_node(x, agg, ub, wp, vp):
    w1, b1, w2, b2 = wp
    wv1, bv1, wv2, bv2 = vp
    full = lambda shape: pl.BlockSpec(shape, lambda i: (0, 0))
    return pl.pallas_call(
        _node_body,
        grid=(_N // _NB,),
        in_specs=[
            pl.BlockSpec((_NB, _D), lambda i: (i, 0)),
            pl.BlockSpec((_NB, _D), lambda i: (i, 0)),
            pl.BlockSpec((_NB, _D), lambda i: (i, 0)),
            full((_D, 768)), full((_D, 768)), full((_D, 768)), full((1, 768)),
            full((768, _D)), full((1, _D)),
            full((_D, 256)), full((1, 256)), full((256, _D)), full((1, _D)),
        ],
        out_specs=[
            pl.BlockSpec((_NB, _D), lambda i: (i, 0)),
            pl.BlockSpec((_NB, _D), lambda i: (i, 0)),
        ],
        out_shape=[
            jax.ShapeDtypeStruct((_N, _D), jnp.float32),
            jax.ShapeDtypeStruct((_N, _D), jnp.float32),
        ],
        compiler_params=pltpu.CompilerParams(
            dimension_semantics=("arbitrary",)),
    )(x, agg, ub,
      w1[0:_D], w1[_D:2 * _D], w1[2 * _D:3 * _D], b1.reshape(1, 768),
      w2, b2.reshape(1, _D),
      wv1, bv1.reshape(1, 256), wv2, bv2.reshape(1, _D))


_CE = 80
_GR = _G // _NC
_SHG = 512
_NCHE = _N // _CE
_ITE = 8
_GZ = _SHG // _NS


@functools.partial(
    pl.kernel,
    mesh=_mesh,
    out_type=[
        jax.ShapeDtypeStruct((_G, _D), jnp.float32),
        jax.ShapeDtypeStruct((_G, _D), jnp.float32),
    ],
    scratch_types=[
        pltpu.VMEM((_CE,), jnp.int32),
        pltpu.VMEM((_CE,), jnp.int32),
        pltpu.VMEM((_CE, _D), jnp.float32),
        pltpu.VMEM((_CE, _D), jnp.float32),
        pltpu.VMEM((_CE, _D), jnp.float32),
        pltpu.VMEM((_CE, _D), jnp.float32),
        pltpu.VMEM((_GZ, _D), jnp.float32),
        pltpu.VMEM((_GZ, _D), jnp.float32),
        pltpu.VMEM_SHARED((_SHG, _D), jnp.float32),
        pltpu.VMEM_SHARED((_SHG, _D), jnp.float32),
        pltpu.VMEM_SHARED((_SHG, _D), jnp.float32),
        pltpu.VMEM_SHARED((_SHG, _D), jnp.float32),
    ],
)
def _sc_scatter_g(xnew_hbm, nsum_hbm, ncnt_hbm, batch_hbm,
                  zrow_hbm, ones_hbm,
                  gx_out, ge_out,
                  bidx, lidx, xv, sv, cv, ones_v, gbuf, cbuf,
                  sgx, sge, sgxc, sgec):
    c = lax.axis_index("c")
    t = lax.axis_index("s")
    gbase = c * _GR
    zb = pl.multiple_of(t * _GZ, 8)
    pltpu.sync_copy(zrow_hbm, sgx.at[pl.ds(zb, _GZ)])
    pltpu.sync_copy(zrow_hbm, sge.at[pl.ds(zb, _GZ)])
    pltpu.sync_copy(zrow_hbm, sgxc.at[pl.ds(zb, _GZ)])
    pltpu.sync_copy(zrow_hbm, sgec.at[pl.ds(zb, _GZ)])
    pltpu.sync_copy(ones_hbm, ones_v)
    plsc.subcore_barrier()

    def chunk(i, carry):
        cid = t + _NS * i

        @pl.when(cid < _NCHE)
        def _():
            b = pl.multiple_of(cid * _CE, 8)
            pltpu.sync_copy(batch_hbm.at[pl.ds(b, _CE)], bidx)

            def ixl(j, c2):
                v = bidx[pl.ds(j * 16, 16)] - gbase
                ok = (v >= 0) & (v < _GR)
                lidx[pl.ds(j * 16, 16)] = jnp.where(ok, v, _GR)
                return c2
            lax.fori_loop(0, _CE // 16, ixl, 0)
            pltpu.sync_copy(xnew_hbm.at[pl.ds(b, _CE)], xv)
            pltpu.sync_copy(nsum_hbm.at[pl.ds(b, _CE)], sv)
            pltpu.sync_copy(ncnt_hbm.at[pl.ds(b, _CE)], cv)
            pltpu.sync_copy(xv, sgx.at[lidx], add=True)
            pltpu.sync_copy(sv, sge.at[lidx], add=True)
            pltpu.sync_copy(ones_v, sgxc.at[lidx], add=True)
            pltpu.sync_copy(cv, sgec.at[lidx], add=True)
        return carry

    lax.fori_loop(0, _ITE, chunk, 0)
    plsc.subcore_barrier()

    @pl.when(t < _GR // _GZ)
    def _():
        gb = pl.multiple_of(gbase + t * _GZ, 8)

        def final(src, cnts, out):
            pltpu.sync_copy(src.at[pl.ds(zb, _GZ)], gbuf)
            pltpu.sync_copy(cnts.at[pl.ds(zb, _GZ)], cbuf)

            def divloop(r, carry):
                inv = 1.0 / jnp.maximum(cbuf[r, pl.ds(0, 16)], 1.0)

                def dj(j, c2):
                    gbuf[r, pl.ds(j * 16, 16)] = gbuf[r, pl.ds(j * 16, 16)] * inv
                    return c2
                lax.fori_loop(0, _D // 16, dj, 0)
                return carry
            lax.fori_loop(0, _GZ, divloop, 0)
            pltpu.sync_copy(gbuf, out.at[pl.ds(gb, _GZ)])

        final(sgx, sgxc, gx_out)
        final(sge, sgec, ge_out)


def _glob_body(u, gx, ge, w1a, w1b, w1c, b1, w2, b2,
               wu1, bu1, wu2, bu2, uo_out):
    h = jnp.dot(u[...], w1a[...], preferred_element_type=jnp.float32)
    h = h + jnp.dot(gx[...], w1b[...], preferred_element_type=jnp.float32)
    h = h + jnp.dot(ge[...], w1c[...], preferred_element_type=jnp.float32)
    h = _leaky(h + b1[...])
    un = _leaky(jnp.dot(h, w2[...], preferred_element_type=jnp.float32) + b2[...])
    h2 = _leaky(jnp.dot(un, wu1[...], preferred_element_type=jnp.float32) + bu1[...])
    uo_out[...] = u[...] + jnp.dot(h2, wu2[...], preferred_element_type=jnp.float32) + bu2[...]


def _tc_glob(u, gx, ge, wp, up):
    w1, b1, w2, b2 = wp
    wu1, bu1, wu2, bu2 = up
    return pl.pallas_call(
        _glob_body,
        out_shape=jax.ShapeDtypeStruct((_G, _D), jnp.float32),
    )(u, gx, ge,
      w1[0:_D], w1[_D:2 * _D], w1[2 * _D:3 * _D], b1.reshape(1, 768),
      w2, b2.reshape(1, _D),
      wu1, bu1.reshape(1, 256), wu2, bu2.reshape(1, _D))


def kernel(x, edge_index, edge_attr, u, batch, params):
    row = edge_index[0].astype(jnp.int32)
    col = edge_index[1].astype(jnp.int32)
    batch32 = batch.astype(jnp.int32)

    ub = _sc_gather_ub(u, batch32)
    xb = x
    e_new = [None, None]
    e_out = [None, None]
    for hlf in range(2):
        r_h = lax.slice(row, (hlf * _E2,), ((hlf + 1) * _E2,))
        c_h = lax.slice(col, (hlf * _E2,), ((hlf + 1) * _E2,))
        ea_h = lax.slice(edge_attr, (hlf * _E2, 0), ((hlf + 1) * _E2, _D))
        xr, xc, ue = _sc_gather_edges(xb, ub, r_h, c_h)
        e_new[hlf], e_out[hlf] = _tc_edge(xr, xc, ea_h, ue,
                                          params['edge'], params['e'])

    zrow = jnp.zeros((_ZR, _D), jnp.float32)
    ones_c = jnp.ones((_CC, _D), jnp.float32)
    agg, nsum, ncnt = _sc_scatter_e(e_new[0], e_new[1], row, zrow, ones_c)

    x_new, x_out = _tc_node(x, agg, ub, params['node'], params['v'])

    zrow_g = jnp.zeros((_GZ, _D), jnp.float32)
    ones_e = jnp.ones((_CE, _D), jnp.float32)
    gx, ge = _sc_scatter_g(x_new, nsum, ncnt, batch32, zrow_g, ones_e)

    u_out = _tc_glob(u, gx, ge, params['glob'], params['u'])
    return (x_out, jnp.concatenate(e_out, axis=0), u_out)


# speedup vs baseline: 1.1748x; 1.1584x over previous
import functools

import jax
import jax.numpy as jnp
from jax import lax
from jax.experimental import pallas as pl
from jax.experimental.pallas import tpu as pltpu
from jax.experimental.pallas import tpu_sc as plsc

_N = 10000
_E = 320000
_G = 512
_D = 128

_NC, _NS = 2, 16
_NW = _NC * _NS

_mesh = plsc.VectorSubcoreMesh(core_axis_name="c", subcore_axis_name="s")

_CA = 400
_E2 = _E // 2
_CAH = 400
_NCHH = _E2 // _CAH
_ITA = 13


@functools.partial(
    pl.kernel,
    mesh=_mesh,
    out_type=jax.ShapeDtypeStruct((_N, _D), jnp.float32),
    scratch_types=[
        pltpu.VMEM((_CA,), jnp.int32),
        pltpu.VMEM((_CA, _D), jnp.float32),
        pltpu.SemaphoreType.DMA,
    ],
)
def _sc_gather_ub(u_hbm, batch_hbm, ub_out, bidx, buf, sem):
    wid = lax.axis_index("c") * _NS + lax.axis_index("s")

    @pl.when(wid < _N // _CA)
    def _():
        nb = pl.multiple_of(wid * _CA, 8)
        pltpu.sync_copy(batch_hbm.at[pl.ds(nb, _CA)], bidx)
        pltpu.async_copy(u_hbm.at[bidx], buf, sem).wait()
        pltpu.sync_copy(buf, ub_out.at[pl.ds(nb, _CA)])


@functools.partial(
    pl.kernel,
    mesh=_mesh,
    out_type=[
        jax.ShapeDtypeStruct((_E2, _D), jnp.float32),
        jax.ShapeDtypeStruct((_E2, _D), jnp.float32),
        jax.ShapeDtypeStruct((_E2, _D), jnp.float32),
    ],
    scratch_types=[
        pltpu.VMEM((_CAH,), jnp.int32),
        pltpu.VMEM((_CAH,), jnp.int32),
        pltpu.VMEM((_CAH, _D), jnp.float32),
        pltpu.VMEM((_CAH, _D), jnp.float32),
        pltpu.SemaphoreType.DMA,
        pltpu.SemaphoreType.DMA,
    ],
)
def _sc_gather_edges(x_hbm, ub_hbm, row_hbm, col_hbm,
                     xr_out, xc_out, ue_out,
                     ridx, cidx, buf_a, buf_b, sem1, sem2):
    wid = lax.axis_index("c") * _NS + lax.axis_index("s")

    def chunk(i, carry):
        cid = wid + _NW * i

        @pl.when(cid < _NCHH)
        def _():
            b = pl.multiple_of(cid * _CAH, 8)
            pltpu.sync_copy(row_hbm.at[pl.ds(b, _CAH)], ridx)
            pltpu.sync_copy(col_hbm.at[pl.ds(b, _CAH)], cidx)
            cp_a = pltpu.async_copy(x_hbm.at[ridx], buf_a, sem1)
            cp_b = pltpu.async_copy(x_hbm.at[cidx], buf_b, sem2)
            cp_a.wait()
            cp_b.wait()
            pltpu.sync_copy(buf_a, xr_out.at[pl.ds(b, _CAH)])
            pltpu.sync_copy(buf_b, xc_out.at[pl.ds(b, _CAH)])
            pltpu.async_copy(ub_hbm.at[ridx], buf_a, sem1).wait()
            pltpu.sync_copy(buf_a, ue_out.at[pl.ds(b, _CAH)])
        return carry

    lax.fori_loop(0, _ITA, chunk, 0)


_TB = 3200


def _leaky(h):
    return jnp.where(h >= 0, h, 0.01 * h)


def _edge_body(xr, xc, ea, ue, w1a, w1b, w1c, b1, w2, b2,
               we1, be1, we2, be2, en_out, eo_out):
    bf = jnp.bfloat16
    s = (xr[...] + xc[...]).astype(bf)
    h = jnp.dot(s, w1a[...], preferred_element_type=jnp.float32)
    h = h + jnp.dot(ea[...].astype(bf), w1b[...], preferred_element_type=jnp.float32)
    h = h + jnp.dot(ue[...].astype(bf), w1c[...], preferred_element_type=jnp.float32)
    h = _leaky(h + b1[...]).astype(bf)
    en = _leaky(jnp.dot(h, w2[...], preferred_element_type=jnp.float32) + b2[...])
    en_out[...] = en
    h2 = _leaky(jnp.dot(en.astype(bf), we1[...], preferred_element_type=jnp.float32) + be1[...]).astype(bf)
    eo_out[...] = ea[...] + jnp.dot(h2, we2[...], preferred_element_type=jnp.float32) + be2[...]


def _tc_edge(xr, xc, ea, ue, wp, ep):
    w1, b1, w2, b2 = wp
    we1, be1, we2, be2 = ep
    full = lambda shape: pl.BlockSpec(shape, lambda i: (0, 0))
    return pl.pallas_call(
        _edge_body,
        grid=(_E2 // _TB,),
        in_specs=[
            pl.BlockSpec((_TB, _D), lambda i: (i, 0)),
            pl.BlockSpec((_TB, _D), lambda i: (i, 0)),
            pl.BlockSpec((_TB, _D), lambda i: (i, 0)),
            pl.BlockSpec((_TB, _D), lambda i: (i, 0)),
            full((_D, 768)), full((_D, 768)), full((_D, 768)), full((1, 768)),
            full((768, _D)), full((1, _D)),
            full((_D, 256)), full((1, 256)), full((256, _D)), full((1, _D)),
        ],
        out_specs=[
            pl.BlockSpec((_TB, _D), lambda i: (i, 0)),
            pl.BlockSpec((_TB, _D), lambda i: (i, 0)),
        ],
        out_shape=[
            jax.ShapeDtypeStruct((_E2, _D), jnp.float32),
            jax.ShapeDtypeStruct((_E2, _D), jnp.float32),
        ],
        compiler_params=pltpu.CompilerParams(
            dimension_semantics=("arbitrary",)),
    )(xr, xc, ea, ue,
      w1[0:_D].astype(jnp.bfloat16), w1[_D:2 * _D].astype(jnp.bfloat16),
      w1[2 * _D:3 * _D].astype(jnp.bfloat16), b1.reshape(1, 768),
      w2.astype(jnp.bfloat16), b2.reshape(1, _D),
      we1.astype(jnp.bfloat16), be1.reshape(1, 256),
      we2.astype(jnp.bfloat16), be2.reshape(1, _D))


_CC = 128
_NR = _N // _NC
_SHN = 5120
_ZR = _SHN // _NS
_FP = 40
_NP = _ZR // _FP
_NCHC = _E2 // _CC
_ITC = 79


@functools.partial(
    pl.kernel,
    mesh=_mesh,
    out_type=[
        jax.ShapeDtypeStruct((_N, _D), jnp.float32),
        jax.ShapeDtypeStruct((_N, _D), jnp.float32),
    ],
    scratch_types=[
        pltpu.VMEM((_CC,), jnp.int32),
        pltpu.VMEM((_CC,), jnp.int32),
        pltpu.VMEM((_CC, _D), jnp.float32),
        pltpu.VMEM((_CC, _D), jnp.float32),
        pltpu.VMEM((_FP, _D), jnp.float32),
        pltpu.VMEM_SHARED((_SHN, _D), jnp.float32),
        pltpu.VMEM_SHARED((_SHN, _D), jnp.float32),
    ],
)
def _sc_scatter_e(enew_hbm, row_hbm, zrow_hbm, ones_hbm,
                  nsum_out, ncnt_out,
                  ridx, lidx, vals, ones_v, fbuf, ssum, scnt):
    c = lax.axis_index("c")
    t = lax.axis_index("s")
    nbase = c * _NR
    sb = pl.multiple_of(t * _ZR, 8)
    pltpu.sync_copy(zrow_hbm, ssum.at[pl.ds(sb, _ZR)])
    pltpu.sync_copy(zrow_hbm, scnt.at[pl.ds(sb, _ZR)])
    pltpu.sync_copy(ones_hbm, ones_v)
    plsc.subcore_barrier()

    def chunk(i, carry):
        cid = t + _NS * i

        @pl.when(cid < _NCHC)
        def _():
            b = pl.multiple_of(cid * _CC, 8)
            pltpu.sync_copy(row_hbm.at[pl.ds(b, _CC)], ridx)

            def ixl(j, c2):
                v = ridx[pl.ds(j * 16, 16)] - nbase
                ok = (v >= 0) & (v < _NR)
                lidx[pl.ds(j * 16, 16)] = jnp.where(ok, v, _NR)
                return c2
            lax.fori_loop(0, _CC // 16, ixl, 0)
            pltpu.sync_copy(enew_hbm.at[pl.ds(b, _CC)], vals)
            pltpu.sync_copy(vals, ssum.at[lidx], add=True)
            pltpu.sync_copy(ones_v, scnt.at[lidx], add=True)
        return carry

    lax.fori_loop(0, _ITC, chunk, 0)
    plsc.subcore_barrier()

    def piece(p, carry):
        @pl.when((t < _NS - 1) | (p < (_NR - (_NS - 1) * _ZR) // _FP))
        def _():
            lb = pl.multiple_of(sb + p * _FP, 8)
            gb = pl.multiple_of(nbase + sb + p * _FP, 8)
            pltpu.sync_copy(ssum.at[pl.ds(lb, _FP)], fbuf)
            pltpu.sync_copy(fbuf, nsum_out.at[pl.ds(gb, _FP)])
            pltpu.sync_copy(scnt.at[pl.ds(lb, _FP)], fbuf)
            pltpu.sync_copy(fbuf, ncnt_out.at[pl.ds(gb, _FP)])
        return carry
    lax.fori_loop(0, _NP, piece, 0)


_NB = 400


def _node_body(x, s0, s1, c0, c1, ub, w1a, w1b, w1c, b1, w2, b2,
               wv1, bv1, wv2, bv2, xn_out, xo_out, ns_out, nc_out):
    nsum = s0[...] + s1[...]
    ncnt = c0[...] + c1[...]
    ns_out[...] = nsum
    nc_out[...] = ncnt
    agg = nsum / jnp.maximum(ncnt, 1.0)
    h = jnp.dot(x[...], w1a[...], preferred_element_type=jnp.float32)
    h = h + jnp.dot(agg, w1b[...], preferred_element_type=jnp.float32)
    h = h + jnp.dot(ub[...], w1c[...], preferred_element_type=jnp.float32)
    h = _leaky(h + b1[...])
    xn = _leaky(jnp.dot(h, w2[...], preferred_element_type=jnp.float32) + b2[...])
    xn_out[...] = xn
    h2 = _leaky(jnp.dot(xn, wv1[...], preferred_element_type=jnp.float32) + bv1[...])
    xo_out[...] = x[...] + jnp.dot(h2, wv2[...], preferred_element_type=jnp.float32) + bv2[...]


def _tc_node(x, s0, s1, c0, c1, ub, wp, vp):
    w1, b1, w2, b2 = wp
    wv1, bv1, wv2, bv2 = vp
    full = lambda shape: pl.BlockSpec(shape, lambda i: (0, 0))
    return pl.pallas_call(
        _node_body,
        grid=(_N // _NB,),
        in_specs=[
            pl.BlockSpec((_NB, _D), lambda i: (i, 0)),
            pl.BlockSpec((_NB, _D), lambda i: (i, 0)),
            pl.BlockSpec((_NB, _D), lambda i: (i, 0)),
            pl.BlockSpec((_NB, _D), lambda i: (i, 0)),
            pl.BlockSpec((_NB, _D), lambda i: (i, 0)),
            pl.BlockSpec((_NB, _D), lambda i: (i, 0)),
            full((_D, 768)), full((_D, 768)), full((_D, 768)), full((1, 768)),
            full((768, _D)), full((1, _D)),
            full((_D, 256)), full((1, 256)), full((256, _D)), full((1, _D)),
        ],
        out_specs=[
            pl.BlockSpec((_NB, _D), lambda i: (i, 0)),
            pl.BlockSpec((_NB, _D), lambda i: (i, 0)),
            pl.BlockSpec((_NB, _D), lambda i: (i, 0)),
            pl.BlockSpec((_NB, _D), lambda i: (i, 0)),
        ],
        out_shape=[
            jax.ShapeDtypeStruct((_N, _D), jnp.float32),
            jax.ShapeDtypeStruct((_N, _D), jnp.float32),
            jax.ShapeDtypeStruct((_N, _D), jnp.float32),
            jax.ShapeDtypeStruct((_N, _D), jnp.float32),
        ],
        compiler_params=pltpu.CompilerParams(
            dimension_semantics=("arbitrary",)),
    )(x, s0, s1, c0, c1, ub,
      w1[0:_D], w1[_D:2 * _D], w1[2 * _D:3 * _D], b1.reshape(1, 768),
      w2, b2.reshape(1, _D),
      wv1, bv1.reshape(1, 256), wv2, bv2.reshape(1, _D))


_CE = 80
_GR = _G // _NC
_SHG = 512
_NCHE = _N // _CE
_ITE = 8
_GZ = _SHG // _NS


@functools.partial(
    pl.kernel,
    mesh=_mesh,
    out_type=[
        jax.ShapeDtypeStruct((_G, _D), jnp.float32),
        jax.ShapeDtypeStruct((_G, _D), jnp.float32),
    ],
    scratch_types=[
        pltpu.VMEM((_CE,), jnp.int32),
        pltpu.VMEM((_CE,), jnp.int32),
        pltpu.VMEM((_CE, _D), jnp.float32),
        pltpu.VMEM((_CE, _D), jnp.float32),
        pltpu.VMEM((_CE, _D), jnp.float32),
        pltpu.VMEM((_CE, _D), jnp.float32),
        pltpu.VMEM((_GZ, _D), jnp.float32),
        pltpu.VMEM((_GZ, _D), jnp.float32),
        pltpu.VMEM_SHARED((_SHG, _D), jnp.float32),
        pltpu.VMEM_SHARED((_SHG, _D), jnp.float32),
        pltpu.VMEM_SHARED((_SHG, _D), jnp.float32),
        pltpu.VMEM_SHARED((_SHG, _D), jnp.float32),
    ],
)
def _sc_scatter_g(xnew_hbm, nsum_hbm, ncnt_hbm, batch_hbm,
                  zrow_hbm, ones_hbm,
                  gx_out, ge_out,
                  bidx, lidx, xv, sv, cv, ones_v, gbuf, cbuf,
                  sgx, sge, sgxc, sgec):
    c = lax.axis_index("c")
    t = lax.axis_index("s")
    gbase = c * _GR
    zb = pl.multiple_of(t * _GZ, 8)
    pltpu.sync_copy(zrow_hbm, sgx.at[pl.ds(zb, _GZ)])
    pltpu.sync_copy(zrow_hbm, sge.at[pl.ds(zb, _GZ)])
    pltpu.sync_copy(zrow_hbm, sgxc.at[pl.ds(zb, _GZ)])
    pltpu.sync_copy(zrow_hbm, sgec.at[pl.ds(zb, _GZ)])
    pltpu.sync_copy(ones_hbm, ones_v)
    plsc.subcore_barrier()

    def chunk(i, carry):
        cid = t + _NS * i

        @pl.when(cid < _NCHE)
        def _():
            b = pl.multiple_of(cid * _CE, 8)
            pltpu.sync_copy(batch_hbm.at[pl.ds(b, _CE)], bidx)

            def ixl(j, c2):
                v = bidx[pl.ds(j * 16, 16)] - gbase
                ok = (v >= 0) & (v < _GR)
                lidx[pl.ds(j * 16, 16)] = jnp.where(ok, v, _GR)
                return c2
            lax.fori_loop(0, _CE // 16, ixl, 0)
            pltpu.sync_copy(xnew_hbm.at[pl.ds(b, _CE)], xv)
            pltpu.sync_copy(nsum_hbm.at[pl.ds(b, _CE)], sv)
            pltpu.sync_copy(ncnt_hbm.at[pl.ds(b, _CE)], cv)
            pltpu.sync_copy(xv, sgx.at[lidx], add=True)
            pltpu.sync_copy(sv, sge.at[lidx], add=True)
            pltpu.sync_copy(ones_v, sgxc.at[lidx], add=True)
            pltpu.sync_copy(cv, sgec.at[lidx], add=True)
        return carry

    lax.fori_loop(0, _ITE, chunk, 0)
    plsc.subcore_barrier()

    @pl.when(t < _GR // _GZ)
    def _():
        gb = pl.multiple_of(gbase + t * _GZ, 8)

        def final(src, cnts, out):
            pltpu.sync_copy(src.at[pl.ds(zb, _GZ)], gbuf)
            pltpu.sync_copy(cnts.at[pl.ds(zb, _GZ)], cbuf)

            def divloop(r, carry):
                inv = 1.0 / jnp.maximum(cbuf[r, pl.ds(0, 16)], 1.0)

                def dj(j, c2):
                    gbuf[r, pl.ds(j * 16, 16)] = gbuf[r, pl.ds(j * 16, 16)] * inv
                    return c2
                lax.fori_loop(0, _D // 16, dj, 0)
                return carry
            lax.fori_loop(0, _GZ, divloop, 0)
            pltpu.sync_copy(gbuf, out.at[pl.ds(gb, _GZ)])

        final(sgx, sgxc, gx_out)
        final(sge, sgec, ge_out)


def _glob_body(u, gx, ge, w1a, w1b, w1c, b1, w2, b2,
               wu1, bu1, wu2, bu2, uo_out):
    h = jnp.dot(u[...], w1a[...], preferred_element_type=jnp.float32)
    h = h + jnp.dot(gx[...], w1b[...], preferred_element_type=jnp.float32)
    h = h + jnp.dot(ge[...], w1c[...], preferred_element_type=jnp.float32)
    h = _leaky(h + b1[...])
    un = _leaky(jnp.dot(h, w2[...], preferred_element_type=jnp.float32) + b2[...])
    h2 = _leaky(jnp.dot(un, wu1[...], preferred_element_type=jnp.float32) + bu1[...])
    uo_out[...] = u[...] + jnp.dot(h2, wu2[...], preferred_element_type=jnp.float32) + bu2[...]


def _tc_glob(u, gx, ge, wp, up):
    w1, b1, w2, b2 = wp
    wu1, bu1, wu2, bu2 = up
    return pl.pallas_call(
        _glob_body,
        out_shape=jax.ShapeDtypeStruct((_G, _D), jnp.float32),
    )(u, gx, ge,
      w1[0:_D], w1[_D:2 * _D], w1[2 * _D:3 * _D], b1.reshape(1, 768),
      w2, b2.reshape(1, _D),
      wu1, bu1.reshape(1, 256), wu2, bu2.reshape(1, _D))


def kernel(x, edge_index, edge_attr, u, batch, params):
    row = edge_index[0].astype(jnp.int32)
    col = edge_index[1].astype(jnp.int32)
    batch32 = batch.astype(jnp.int32)

    ub = _sc_gather_ub(u, batch32)
    xb = x
    e_new = [None, None]
    e_out = [None, None]
    for hlf in range(2):
        r_h = lax.slice(row, (hlf * _E2,), ((hlf + 1) * _E2,))
        c_h = lax.slice(col, (hlf * _E2,), ((hlf + 1) * _E2,))
        ea_h = lax.slice(edge_attr, (hlf * _E2, 0), ((hlf + 1) * _E2, _D))
        xr, xc, ue = _sc_gather_edges(xb, ub, r_h, c_h)
        e_new[hlf], e_out[hlf] = _tc_edge(xr, xc, ea_h, ue,
                                          params['edge'], params['e'])

    zrow = jnp.zeros((_ZR, _D), jnp.float32)
    ones_c = jnp.ones((_CC, _D), jnp.float32)
    s0, c0 = _sc_scatter_e(e_new[0], lax.slice(row, (0,), (_E2,)),
                           zrow, ones_c)
    s1, c1 = _sc_scatter_e(e_new[1], lax.slice(row, (_E2,), (_E,)),
                           zrow, ones_c)

    x_new, x_out, nsum, ncnt = _tc_node(x, s0, s1, c0, c1, ub,
                                        params['node'], params['v'])

    zrow_g = jnp.zeros((_GZ, _D), jnp.float32)
    ones_e = jnp.ones((_CE, _D), jnp.float32)
    gx, ge = _sc_scatter_g(x_new, nsum, ncnt, batch32, zrow_g, ones_e)

    u_out = _tc_glob(u, gx, ge, params['glob'], params['u'])
    return (x_out, jnp.concatenate(e_out, axis=0), u_out)


# speedup vs baseline: 1.1804x; 1.0048x over previous
import functools

import jax
import jax.numpy as jnp
from jax import lax
from jax.experimental import pallas as pl
from jax.experimental.pallas import tpu as pltpu
from jax.experimental.pallas import tpu_sc as plsc

_N = 10000
_E = 320000
_G = 512
_D = 128

_NC, _NS = 2, 16
_NW = _NC * _NS

_mesh = plsc.VectorSubcoreMesh(core_axis_name="c", subcore_axis_name="s")

_CA = 400
_E2 = _E // 4
_CAH = 400
_NCHH = _E2 // _CAH
_ITA = 7


@functools.partial(
    pl.kernel,
    mesh=_mesh,
    out_type=jax.ShapeDtypeStruct((_N, _D), jnp.float32),
    scratch_types=[
        pltpu.VMEM((_CA,), jnp.int32),
        pltpu.VMEM((_CA, _D), jnp.float32),
        pltpu.SemaphoreType.DMA,
    ],
)
def _sc_gather_ub(u_hbm, batch_hbm, ub_out, bidx, buf, sem):
    wid = lax.axis_index("c") * _NS + lax.axis_index("s")

    @pl.when(wid < _N // _CA)
    def _():
        nb = pl.multiple_of(wid * _CA, 8)
        pltpu.sync_copy(batch_hbm.at[pl.ds(nb, _CA)], bidx)
        pltpu.async_copy(u_hbm.at[bidx], buf, sem).wait()
        pltpu.sync_copy(buf, ub_out.at[pl.ds(nb, _CA)])


@functools.partial(
    pl.kernel,
    mesh=_mesh,
    out_type=[
        jax.ShapeDtypeStruct((_E2, _D), jnp.float32),
        jax.ShapeDtypeStruct((_E2, _D), jnp.float32),
        jax.ShapeDtypeStruct((_E2, _D), jnp.float32),
    ],
    scratch_types=[
        pltpu.VMEM((_CAH,), jnp.int32),
        pltpu.VMEM((_CAH,), jnp.int32),
        pltpu.VMEM((_CAH, _D), jnp.float32),
        pltpu.VMEM((_CAH, _D), jnp.float32),
        pltpu.SemaphoreType.DMA,
        pltpu.SemaphoreType.DMA,
    ],
)
def _sc_gather_edges(x_hbm, ub_hbm, row_hbm, col_hbm,
                     xr_out, xc_out, ue_out,
                     ridx, cidx, buf_a, buf_b, sem1, sem2):
    wid = lax.axis_index("c") * _NS + lax.axis_index("s")

    def chunk(i, carry):
        cid = wid + _NW * i

        @pl.when(cid < _NCHH)
        def _():
            b = pl.multiple_of(cid * _CAH, 8)
            pltpu.sync_copy(row_hbm.at[pl.ds(b, _CAH)], ridx)
            pltpu.sync_copy(col_hbm.at[pl.ds(b, _CAH)], cidx)
            cp_a = pltpu.async_copy(x_hbm.at[ridx], buf_a, sem1)
            cp_b = pltpu.async_copy(x_hbm.at[cidx], buf_b, sem2)
            cp_a.wait()
            cp_b.wait()
            pltpu.sync_copy(buf_a, xr_out.at[pl.ds(b, _CAH)])
            pltpu.sync_copy(buf_b, xc_out.at[pl.ds(b, _CAH)])
            pltpu.async_copy(ub_hbm.at[ridx], buf_a, sem1).wait()
            pltpu.sync_copy(buf_a, ue_out.at[pl.ds(b, _CAH)])
        return carry

    lax.fori_loop(0, _ITA, chunk, 0)


_TB = 3200


def _leaky(h):
    return jnp.where(h >= 0, h, 0.01 * h)


def _edge_body(xr, xc, ea, ue, w1a, w1b, w1c, b1, w2, b2,
               we1, be1, we2, be2, en_out, eo_out):
    bf = jnp.bfloat16
    s = (xr[...] + xc[...]).astype(bf)
    h = jnp.dot(s, w1a[...], preferred_element_type=jnp.float32)
    h = h + jnp.dot(ea[...].astype(bf), w1b[...], preferred_element_type=jnp.float32)
    h = h + jnp.dot(ue[...].astype(bf), w1c[...], preferred_element_type=jnp.float32)
    h = _leaky(h + b1[...]).astype(bf)
    en = _leaky(jnp.dot(h, w2[...], preferred_element_type=jnp.float32) + b2[...])
    en_out[...] = en
    h2 = _leaky(jnp.dot(en.astype(bf), we1[...], preferred_element_type=jnp.float32) + be1[...]).astype(bf)
    eo_out[...] = ea[...] + jnp.dot(h2, we2[...], preferred_element_type=jnp.float32) + be2[...]


def _tc_edge(xr, xc, ea, ue, wp, ep):
    w1, b1, w2, b2 = wp
    we1, be1, we2, be2 = ep
    full = lambda shape: pl.BlockSpec(shape, lambda i: (0, 0))
    return pl.pallas_call(
        _edge_body,
        grid=(_E2 // _TB,),
        in_specs=[
            pl.BlockSpec((_TB, _D), lambda i: (i, 0)),
            pl.BlockSpec((_TB, _D), lambda i: (i, 0)),
            pl.BlockSpec((_TB, _D), lambda i: (i, 0)),
            pl.BlockSpec((_TB, _D), lambda i: (i, 0)),
            full((_D, 768)), full((_D, 768)), full((_D, 768)), full((1, 768)),
            full((768, _D)), full((1, _D)),
            full((_D, 256)), full((1, 256)), full((256, _D)), full((1, _D)),
        ],
        out_specs=[
            pl.BlockSpec((_TB, _D), lambda i: (i, 0)),
            pl.BlockSpec((_TB, _D), lambda i: (i, 0)),
        ],
        out_shape=[
            jax.ShapeDtypeStruct((_E2, _D), jnp.float32),
            jax.ShapeDtypeStruct((_E2, _D), jnp.float32),
        ],
        compiler_params=pltpu.CompilerParams(
            dimension_semantics=("arbitrary",)),
    )(xr, xc, ea, ue,
      w1[0:_D].astype(jnp.bfloat16), w1[_D:2 * _D].astype(jnp.bfloat16),
      w1[2 * _D:3 * _D].astype(jnp.bfloat16), b1.reshape(1, 768),
      w2.astype(jnp.bfloat16), b2.reshape(1, _D),
      we1.astype(jnp.bfloat16), be1.reshape(1, 256),
      we2.astype(jnp.bfloat16), be2.reshape(1, _D))


_CC = 128
_NR = _N // _NC
_SHN = 5120
_ZR = _SHN // _NS
_FP = 40
_NP = _ZR // _FP
_NCHC = _E2 // _CC
_ITC = 40


@functools.partial(
    pl.kernel,
    mesh=_mesh,
    out_type=[
        jax.ShapeDtypeStruct((_N, _D), jnp.float32),
        jax.ShapeDtypeStruct((_N, _D), jnp.float32),
    ],
    scratch_types=[
        pltpu.VMEM((_CC,), jnp.int32),
        pltpu.VMEM((_CC,), jnp.int32),
        pltpu.VMEM((_CC, _D), jnp.float32),
        pltpu.VMEM((_CC, _D), jnp.float32),
        pltpu.VMEM((_FP, _D), jnp.float32),
        pltpu.VMEM_SHARED((_SHN, _D), jnp.float32),
        pltpu.VMEM_SHARED((_SHN, _D), jnp.float32),
    ],
)
def _sc_scatter_e(enew_hbm, row_hbm, zrow_hbm, ones_hbm,
                  nsum_out, ncnt_out,
                  ridx, lidx, vals, ones_v, fbuf, ssum, scnt):
    c = lax.axis_index("c")
    t = lax.axis_index("s")
    nbase = c * _NR
    sb = pl.multiple_of(t * _ZR, 8)
    pltpu.sync_copy(zrow_hbm, ssum.at[pl.ds(sb, _ZR)])
    pltpu.sync_copy(zrow_hbm, scnt.at[pl.ds(sb, _ZR)])
    pltpu.sync_copy(ones_hbm, ones_v)
    plsc.subcore_barrier()

    def chunk(i, carry):
        cid = t + _NS * i

        @pl.when(cid < _NCHC)
        def _():
            b = pl.multiple_of(cid * _CC, 8)
            pltpu.sync_copy(row_hbm.at[pl.ds(b, _CC)], ridx)

            def ixl(j, c2):
                v = ridx[pl.ds(j * 16, 16)] - nbase
                ok = (v >= 0) & (v < _NR)
                lidx[pl.ds(j * 16, 16)] = jnp.where(ok, v, _NR)
                return c2
            lax.fori_loop(0, _CC // 16, ixl, 0)
            pltpu.sync_copy(enew_hbm.at[pl.ds(b, _CC)], vals)
            pltpu.sync_copy(vals, ssum.at[lidx], add=True)
            pltpu.sync_copy(ones_v, scnt.at[lidx], add=True)
        return carry

    lax.fori_loop(0, _ITC, chunk, 0)
    plsc.subcore_barrier()

    def piece(p, carry):
        @pl.when((t < _NS - 1) | (p < (_NR - (_NS - 1) * _ZR) // _FP))
        def _():
            lb = pl.multiple_of(sb + p * _FP, 8)
            gb = pl.multiple_of(nbase + sb + p * _FP, 8)
            pltpu.sync_copy(ssum.at[pl.ds(lb, _FP)], fbuf)
            pltpu.sync_copy(fbuf, nsum_out.at[pl.ds(gb, _FP)])
            pltpu.sync_copy(scnt.at[pl.ds(lb, _FP)], fbuf)
            pltpu.sync_copy(fbuf, ncnt_out.at[pl.ds(gb, _FP)])
        return carry
    lax.fori_loop(0, _NP, piece, 0)


_NB = 400


def _node_body(x, s0, s1, s2, s3, c0, c1, c2, c3, ub,
               w1a, w1b, w1c, b1, w2, b2,
               wv1, bv1, wv2, bv2, xn_out, xo_out, ns_out, nc_out):
    nsum = s0[...] + s1[...] + s2[...] + s3[...]
    ncnt = c0[...] + c1[...] + c2[...] + c3[...]
    ns_out[...] = nsum
    nc_out[...] = ncnt
    agg = nsum / jnp.maximum(ncnt, 1.0)
    h = jnp.dot(x[...], w1a[...], preferred_element_type=jnp.float32)
    h = h + jnp.dot(agg, w1b[...], preferred_element_type=jnp.float32)
    h = h + jnp.dot(ub[...], w1c[...], preferred_element_type=jnp.float32)
    h = _leaky(h + b1[...])
    xn = _leaky(jnp.dot(h, w2[...], preferred_element_type=jnp.float32) + b2[...])
    xn_out[...] = xn
    h2 = _leaky(jnp.dot(xn, wv1[...], preferred_element_type=jnp.float32) + bv1[...])
    xo_out[...] = x[...] + jnp.dot(h2, wv2[...], preferred_element_type=jnp.float32) + bv2[...]


def _tc_node(x, ss, cc, ub, wp, vp):
    w1, b1, w2, b2 = wp
    wv1, bv1, wv2, bv2 = vp
    full = lambda shape: pl.BlockSpec(shape, lambda i: (0, 0))
    return pl.pallas_call(
        _node_body,
        grid=(_N // _NB,),
        in_specs=[
            pl.BlockSpec((_NB, _D), lambda i: (i, 0)),
            pl.BlockSpec((_NB, _D), lambda i: (i, 0)),
            pl.BlockSpec((_NB, _D), lambda i: (i, 0)),
            pl.BlockSpec((_NB, _D), lambda i: (i, 0)),
            pl.BlockSpec((_NB, _D), lambda i: (i, 0)),
            pl.BlockSpec((_NB, _D), lambda i: (i, 0)),
            pl.BlockSpec((_NB, _D), lambda i: (i, 0)),
            pl.BlockSpec((_NB, _D), lambda i: (i, 0)),
            pl.BlockSpec((_NB, _D), lambda i: (i, 0)),
            pl.BlockSpec((_NB, _D), lambda i: (i, 0)),
            full((_D, 768)), full((_D, 768)), full((_D, 768)), full((1, 768)),
            full((768, _D)), full((1, _D)),
            full((_D, 256)), full((1, 256)), full((256, _D)), full((1, _D)),
        ],
        out_specs=[
            pl.BlockSpec((_NB, _D), lambda i: (i, 0)),
            pl.BlockSpec((_NB, _D), lambda i: (i, 0)),
            pl.BlockSpec((_NB, _D), lambda i: (i, 0)),
            pl.BlockSpec((_NB, _D), lambda i: (i, 0)),
        ],
        out_shape=[
            jax.ShapeDtypeStruct((_N, _D), jnp.float32),
            jax.ShapeDtypeStruct((_N, _D), jnp.float32),
            jax.ShapeDtypeStruct((_N, _D), jnp.float32),
            jax.ShapeDtypeStruct((_N, _D), jnp.float32),
        ],
        compiler_params=pltpu.CompilerParams(
            dimension_semantics=("arbitrary",)),
    )(x, ss[0], ss[1], ss[2], ss[3], cc[0], cc[1], cc[2], cc[3], ub,
      w1[0:_D], w1[_D:2 * _D], w1[2 * _D:3 * _D], b1.reshape(1, 768),
      w2, b2.reshape(1, _D),
      wv1, bv1.reshape(1, 256), wv2, bv2.reshape(1, _D))


_CE = 80
_GR = _G // _NC
_SHG = 512
_NCHE = _N // _CE
_ITE = 8
_GZ = _SHG // _NS


@functools.partial(
    pl.kernel,
    mesh=_mesh,
    out_type=[
        jax.ShapeDtypeStruct((_G, _D), jnp.float32),
        jax.ShapeDtypeStruct((_G, _D), jnp.float32),
    ],
    scratch_types=[
        pltpu.VMEM((_CE,), jnp.int32),
        pltpu.VMEM((_CE,), jnp.int32),
        pltpu.VMEM((_CE, _D), jnp.float32),
        pltpu.VMEM((_CE, _D), jnp.float32),
        pltpu.VMEM((_CE, _D), jnp.float32),
        pltpu.VMEM((_CE, _D), jnp.float32),
        pltpu.VMEM((_GZ, _D), jnp.float32),
        pltpu.VMEM((_GZ, _D), jnp.float32),
        pltpu.VMEM_SHARED((_SHG, _D), jnp.float32),
        pltpu.VMEM_SHARED((_SHG, _D), jnp.float32),
        pltpu.VMEM_SHARED((_SHG, _D), jnp.float32),
        pltpu.VMEM_SHARED((_SHG, _D), jnp.float32),
    ],
)
def _sc_scatter_g(xnew_hbm, nsum_hbm, ncnt_hbm, batch_hbm,
                  zrow_hbm, ones_hbm,
                  gx_out, ge_out,
                  bidx, lidx, xv, sv, cv, ones_v, gbuf, cbuf,
                  sgx, sge, sgxc, sgec):
    c = lax.axis_index("c")
    t = lax.axis_index("s")
    gbase = c * _GR
    zb = pl.multiple_of(t * _GZ, 8)
    pltpu.sync_copy(zrow_hbm, sgx.at[pl.ds(zb, _GZ)])
    pltpu.sync_copy(zrow_hbm, sge.at[pl.ds(zb, _GZ)])
    pltpu.sync_copy(zrow_hbm, sgxc.at[pl.ds(zb, _GZ)])
    pltpu.sync_copy(zrow_hbm, sgec.at[pl.ds(zb, _GZ)])
    pltpu.sync_copy(ones_hbm, ones_v)
    plsc.subcore_barrier()

    def chunk(i, carry):
        cid = t + _NS * i

        @pl.when(cid < _NCHE)
        def _():
            b = pl.multiple_of(cid * _CE, 8)
            pltpu.sync_copy(batch_hbm.at[pl.ds(b, _CE)], bidx)

            def ixl(j, c2):
                v = bidx[pl.ds(j * 16, 16)] - gbase
                ok = (v >= 0) & (v < _GR)
                lidx[pl.ds(j * 16, 16)] = jnp.where(ok, v, _GR)
                return c2
            lax.fori_loop(0, _CE // 16, ixl, 0)
            pltpu.sync_copy(xnew_hbm.at[pl.ds(b, _CE)], xv)
            pltpu.sync_copy(nsum_hbm.at[pl.ds(b, _CE)], sv)
            pltpu.sync_copy(ncnt_hbm.at[pl.ds(b, _CE)], cv)
            pltpu.sync_copy(xv, sgx.at[lidx], add=True)
            pltpu.sync_copy(sv, sge.at[lidx], add=True)
            pltpu.sync_copy(ones_v, sgxc.at[lidx], add=True)
            pltpu.sync_copy(cv, sgec.at[lidx], add=True)
        return carry

    lax.fori_loop(0, _ITE, chunk, 0)
    plsc.subcore_barrier()

    @pl.when(t < _GR // _GZ)
    def _():
        gb = pl.multiple_of(gbase + t * _GZ, 8)

        def final(src, cnts, out):
            pltpu.sync_copy(src.at[pl.ds(zb, _GZ)], gbuf)
            pltpu.sync_copy(cnts.at[pl.ds(zb, _GZ)], cbuf)

            def divloop(r, carry):
                inv = 1.0 / jnp.maximum(cbuf[r, pl.ds(0, 16)], 1.0)

                def dj(j, c2):
                    gbuf[r, pl.ds(j * 16, 16)] = gbuf[r, pl.ds(j * 16, 16)] * inv
                    return c2
                lax.fori_loop(0, _D // 16, dj, 0)
                return carry
            lax.fori_loop(0, _GZ, divloop, 0)
            pltpu.sync_copy(gbuf, out.at[pl.ds(gb, _GZ)])

        final(sgx, sgxc, gx_out)
        final(sge, sgec, ge_out)


def _glob_body(u, gx, ge, w1a, w1b, w1c, b1, w2, b2,
               wu1, bu1, wu2, bu2, uo_out):
    h = jnp.dot(u[...], w1a[...], preferred_element_type=jnp.float32)
    h = h + jnp.dot(gx[...], w1b[...], preferred_element_type=jnp.float32)
    h = h + jnp.dot(ge[...], w1c[...], preferred_element_type=jnp.float32)
    h = _leaky(h + b1[...])
    un = _leaky(jnp.dot(h, w2[...], preferred_element_type=jnp.float32) + b2[...])
    h2 = _leaky(jnp.dot(un, wu1[...], preferred_element_type=jnp.float32) + bu1[...])
    uo_out[...] = u[...] + jnp.dot(h2, wu2[...], preferred_element_type=jnp.float32) + bu2[...]


def _tc_glob(u, gx, ge, wp, up):
    w1, b1, w2, b2 = wp
    wu1, bu1, wu2, bu2 = up
    return pl.pallas_call(
        _glob_body,
        out_shape=jax.ShapeDtypeStruct((_G, _D), jnp.float32),
    )(u, gx, ge,
      w1[0:_D], w1[_D:2 * _D], w1[2 * _D:3 * _D], b1.reshape(1, 768),
      w2, b2.reshape(1, _D),
      wu1, bu1.reshape(1, 256), wu2, bu2.reshape(1, _D))


def kernel(x, edge_index, edge_attr, u, batch, params):
    row = edge_index[0].astype(jnp.int32)
    col = edge_index[1].astype(jnp.int32)
    batch32 = batch.astype(jnp.int32)

    ub = _sc_gather_ub(u, batch32)
    zrow = jnp.zeros((_ZR, _D), jnp.float32)
    ones_c = jnp.ones((_CC, _D), jnp.float32)
    e_out = []
    ss = []
    cc = []
    for seg in range(4):
        r_h = lax.slice(row, (seg * _E2,), ((seg + 1) * _E2,))
        c_h = lax.slice(col, (seg * _E2,), ((seg + 1) * _E2,))
        ea_h = lax.slice(edge_attr, (seg * _E2, 0), ((seg + 1) * _E2, _D))
        xr, xc, ue = _sc_gather_edges(x, ub, r_h, c_h)
        e_new, eo = _tc_edge(xr, xc, ea_h, ue, params['edge'], params['e'])
        e_out.append(eo)
        s_p, c_p = _sc_scatter_e(e_new, r_h, zrow, ones_c)
        ss.append(s_p)
        cc.append(c_p)

    x_new, x_out, nsum, ncnt = _tc_node(x, ss, cc, ub,
                                        params['node'], params['v'])

    zrow_g = jnp.zeros((_GZ, _D), jnp.float32)
    ones_e = jnp.ones((_CE, _D), jnp.float32)
    gx, ge = _sc_scatter_g(x_new, nsum, ncnt, batch32, zrow_g, ones_e)

    u_out = _tc_glob(u, gx, ge, params['glob'], params['u'])
    return (x_out, jnp.concatenate(e_out, axis=0), u_out)
